# baseline (speedup 1.0000x reference)
import functools

import jax
import jax.numpy as jnp
from jax import lax
from jax.experimental import pallas as pl
from jax.experimental.pallas import tpu as pltpu
from jax.experimental.pallas import tpu_sc as plsc

POP = 8192
G = 256
NSURV = 64
NW = 32
RPW = POP // NW
CH = 64
NCH = RPW // CH


_BLK = 512


def _rank_body(sn_row_ref, sn_col_ref, surv_ref):
    b = pl.program_id(0)

    @pl.when(b == 0)
    def _init():
        surv_ref[...] = jnp.zeros_like(surv_ref)

    sn_all = sn_row_ref[0, :]
    s_blk = sn_col_ref[:, 0]
    jidx = lax.broadcasted_iota(jnp.int32, (_BLK, POP), 1)
    iglob = b * _BLK + lax.broadcasted_iota(jnp.int32, (_BLK, POP), 0)
    gt = sn_all[None, :] > s_blk[:, None]
    eqlt = (sn_all[None, :] == s_blk[:, None]) & (jidx < iglob)
    rank = jnp.sum(jnp.where(gt | eqlt, 1.0, 0.0), axis=1)
    rvec = lax.broadcasted_iota(jnp.int32, (128, _BLK), 0).astype(jnp.float32)
    hit = jnp.where(rank[None, :] == rvec, 1.0, 0.0)
    ivals = (b * _BLK
             + lax.broadcasted_iota(jnp.int32, (128, _BLK), 1)).astype(
                 jnp.float32)
    surv_ref[...] += jnp.sum(hit * ivals, axis=1, keepdims=True).reshape(1, 128)


def _survivor_indices(sn):
    surv = pl.pallas_call(
        _rank_body,
        grid=(POP // _BLK,),
        in_specs=[
            pl.BlockSpec((1, POP), lambda b: (0, 0)),
            pl.BlockSpec((_BLK, 1), lambda b: (b, 0)),
        ],
        out_specs=pl.BlockSpec((1, 128), lambda b: (0, 0)),
        out_shape=jax.ShapeDtypeStruct((1, 128), jnp.float32),
    )(sn.reshape(1, POP), sn.reshape(POP, 1))
    return surv[0, :NSURV].astype(jnp.int32)


_STEPS = (4096, 2048, 1024, 512, 256, 128, 64, 32, 16, 8, 4, 2, 1)


def _count_below(acc_v, q):
    pos = jnp.zeros((16,), jnp.int32)
    for s in _STEPS:
        cand = pos + s
        val = plsc.load_gather(acc_v, [cand - 1])
        pos = jnp.where(val < q, cand, pos)
    cnt = jnp.maximum(pos - 8, 0)
    for o in range(16):
        j = pos + (o - 8)
        jc = jnp.clip(j, 0, POP - 1)
        av = plsc.load_gather(acc_v, [jc])
        cnt = cnt + jnp.where((j >= 0) & (av < q), 1, 0)
    return cnt


def _rsqrt16(x):
    xi = plsc.bitcast(x, jnp.int32)
    y = plsc.bitcast(jnp.int32(0x5F3759DF) - (xi >> 1), jnp.float32)
    for _ in range(3):
        y = y * (1.5 - 0.5 * x * y * y)
    return y


def _sc_body(pop, acc, loc1, loc2, cf, mu, mv, spread, mutch, mask01, surv,
             out, acc_v, l1v, l2v, cfv, muv, i1v, i2v, p1v, p2v, mvv, outv,
             spv, mcv, mkv, survv, elitev, sem):
    wid = lax.axis_index("s") * 2 + lax.axis_index("c")
    base = wid * RPW
    pltpu.sync_copy(acc, acc_v)
    pltpu.sync_copy(spread, spv)
    pltpu.sync_copy(mutch, mcv)

    for k in range(NCH):
        rbase = base + k * CH
        pltpu.sync_copy(loc1.at[pl.ds(rbase, CH)], l1v)
        pltpu.sync_copy(loc2.at[pl.ds(rbase, CH)], l2v)
        pltpu.sync_copy(cf.at[pl.ds(rbase, CH)], cfv)
        pltpu.sync_copy(mu.at[pl.ds(rbase, CH)], muv)
        pltpu.sync_copy(mv.at[pl.ds(rbase, CH)], mvv)
        for g in range(CH // 16):
            sl = pl.ds(g * 16, 16)
            i1v[sl] = _count_below(acc_v, l1v[sl])
            i2v[sl] = _count_below(acc_v, l2v[sl])
        pltpu.async_copy(pop.at[i1v], p1v, sem).wait()
        pltpu.async_copy(pop.at[i2v], p2v, sem).wait()

        spl = spv[:]
        mcl = mcv[:]

        def row_fn(r, _):
            rv = jnp.full((16,), r, jnp.int32)
            kf = (plsc.load_gather(cfv, [rv]) - 0.5) * spl
            mur = plsc.load_gather(muv, [rv])
            ssum = jnp.zeros((16,), jnp.float32)
            for c in range(G // 16):
                mvc = mvv[r, pl.ds(c * 16, 16)]
                ssum = ssum + mvc * mvc
            tot = jnp.sum(ssum)
            tv = jnp.full((16,), tot)
            mscale = jnp.where(mur <= mcl, _rsqrt16(tv) * spl, 0.0)
            for c in range(G // 16):
                cs = pl.ds(c * 16, 16)
                p1c = p1v[r, cs]
                p2c = p2v[r, cs]
                outv[r, cs] = p1c + (p2c - p1c) * kf + mvv[r, cs] * mscale
            return 0

        lax.fori_loop(0, CH, row_fn, 0)

        if k == 0:
            @pl.when(wid == 0)
            def _elite():
                pltpu.sync_copy(surv, survv)
                pltpu.sync_copy(mask01, mkv)
                pltpu.async_copy(pop.at[survv], elitev, sem).wait()

                def blend_fn(r, _):
                    rv = jnp.full((16,), r, jnp.int32)
                    m = plsc.load_gather(mkv, [rv])
                    for c in range(G // 16):
                        cs = pl.ds(c * 16, 16)
                        outv[r, cs] = elitev[r, cs] * m + outv[r, cs] * (1.0 - m)
                    return 0

                lax.fori_loop(0, NSURV, blend_fn, 0)

        pltpu.sync_copy(outv, out.at[pl.ds(rbase, CH)])


@functools.partial(
    pl.kernel,
    mesh=plsc.VectorSubcoreMesh(core_axis_name="c", subcore_axis_name="s"),
    out_type=jax.ShapeDtypeStruct((POP, G), jnp.float32),
    compiler_params=pltpu.CompilerParams(needs_layout_passes=False),
    scratch_types=[
        pltpu.VMEM((POP,), jnp.float32),
        pltpu.VMEM((CH,), jnp.float32),
        pltpu.VMEM((CH,), jnp.float32),
        pltpu.VMEM((CH,), jnp.float32),
        pltpu.VMEM((CH,), jnp.float32),
        pltpu.VMEM((CH,), jnp.int32),
        pltpu.VMEM((CH,), jnp.int32),
        pltpu.VMEM((CH, G), jnp.float32),
        pltpu.VMEM((CH, G), jnp.float32),
        pltpu.VMEM((CH, G), jnp.float32),
        pltpu.VMEM((CH, G), jnp.float32),
        pltpu.VMEM((16,), jnp.float32),
        pltpu.VMEM((16,), jnp.float32),
        pltpu.VMEM((NSURV,), jnp.float32),
        pltpu.VMEM((NSURV,), jnp.int32),
        pltpu.VMEM((NSURV, G), jnp.float32),
        pltpu.SemaphoreType.DMA,
    ],
)
def _sc_kernel(pop, acc, loc1, loc2, cf, mu, mv, spread, mutch, mask01, surv,
               out, *scratch):
    _sc_body(pop, acc, loc1, loc2, cf, mu, mv, spread, mutch, mask01, surv,
             out, *scratch)


def kernel(population, scores, loc1, loc2, cf, mut_u, mut_vec, generation,
           n_survivors):
    scores_n = scores / jnp.sum(scores)
    acc = jnp.cumsum(scores_n)
    acc = acc.at[-1].set(1.0)
    gen = jnp.asarray(generation, jnp.float32)
    spread = jnp.exp(-gen / 10.0)
    mut_ch = 0.1 * jnp.exp(-gen / 10.0)
    spread_v = jnp.full((16,), spread, jnp.float32)
    mutch_v = jnp.full((16,), mut_ch, jnp.float32)
    mask01 = (jnp.arange(NSURV) < n_survivors).astype(jnp.float32)
    surv = _survivor_indices(scores_n)
    return _sc_kernel(
        population, acc,
        loc1.reshape(POP), loc2.reshape(POP),
        cf.reshape(POP), mut_u.reshape(POP), mut_vec,
        spread_v, mutch_v, mask01, surv,
    )


# speedup vs baseline: 1.1464x; 1.1464x over previous
import functools

import jax
import jax.numpy as jnp
from jax import lax
from jax.experimental import pallas as pl
from jax.experimental.pallas import tpu as pltpu
from jax.experimental.pallas import tpu_sc as plsc

POP = 8192
G = 256
NSURV = 64
NW = 32
RPW = POP // NW
CH = 64
NCH = RPW // CH


_BLK = 512
_NCAND = (POP // _BLK) * NSURV


def _rank_local(s):
    n = s.shape[0]
    jidx = lax.broadcasted_iota(jnp.int32, (n, n), 1)
    iidx = lax.broadcasted_iota(jnp.int32, (n, n), 0)
    gt = s[None, :] > s[:, None]
    eqlt = (s[None, :] == s[:, None]) & (jidx < iidx)
    return jnp.sum(jnp.where(gt | eqlt, 1.0, 0.0), axis=1)


def _onehot_compact(rank, vals, nsel):
    rvec = lax.broadcasted_iota(jnp.int32, (nsel, rank.shape[0]), 0)
    hit = jnp.where(rank[None, :] == rvec.astype(jnp.float32), 1.0, 0.0)
    return jnp.sum(hit * vals[None, :], axis=1)


def _stage1_body(sn_ref, cval_ref, cidx_ref):
    b = pl.program_id(0)
    s = sn_ref[0, :]
    lrank = _rank_local(s)
    ivals = (b * _BLK
             + lax.broadcasted_iota(jnp.int32, (1, _BLK), 1)[0]).astype(
                 jnp.float32)
    cval_ref[...] = _onehot_compact(lrank, s, NSURV).reshape(1, 1, NSURV)
    cidx_ref[...] = _onehot_compact(lrank, ivals, NSURV).reshape(1, 1, NSURV)


def _stage2_body(cval_ref, cidx_ref, surv_ref):
    v = cval_ref[0, :]
    ix = cidx_ref[0, :]
    gt = v[None, :] > v[:, None]
    eqlt = (v[None, :] == v[:, None]) & (ix[None, :] < ix[:, None])
    grank = jnp.sum(jnp.where(gt | eqlt, 1.0, 0.0), axis=1)
    surv_ref[...] = _onehot_compact(grank, ix, 128).reshape(1, 128)


def _survivor_indices(sn):
    nblk = POP // _BLK
    cval, cidx = pl.pallas_call(
        _stage1_body,
        grid=(nblk,),
        in_specs=[pl.BlockSpec((1, _BLK), lambda b: (0, b))],
        out_specs=[
            pl.BlockSpec((1, 1, NSURV), lambda b: (b, 0, 0)),
            pl.BlockSpec((1, 1, NSURV), lambda b: (b, 0, 0)),
        ],
        out_shape=[
            jax.ShapeDtypeStruct((nblk, 1, NSURV), jnp.float32),
            jax.ShapeDtypeStruct((nblk, 1, NSURV), jnp.float32),
        ],
    )(sn.reshape(1, POP))
    surv = pl.pallas_call(
        _stage2_body,
        out_shape=jax.ShapeDtypeStruct((1, 128), jnp.float32),
    )(cval.reshape(1, _NCAND), cidx.reshape(1, _NCAND))
    return surv[0, :NSURV].astype(jnp.int32)


_STEPS = (4096, 2048, 1024, 512, 256, 128, 64, 32, 16, 8, 4, 2, 1)


def _count_below(acc_v, q):
    pos = jnp.zeros((16,), jnp.int32)
    for s in _STEPS:
        cand = pos + s
        val = plsc.load_gather(acc_v, [cand - 1])
        pos = jnp.where(val < q, cand, pos)
    cnt = jnp.maximum(pos - 8, 0)
    for o in range(16):
        j = pos + (o - 8)
        jc = jnp.clip(j, 0, POP - 1)
        av = plsc.load_gather(acc_v, [jc])
        cnt = cnt + jnp.where((j >= 0) & (av < q), 1, 0)
    return cnt


def _rsqrt16(x):
    xi = plsc.bitcast(x, jnp.int32)
    y = plsc.bitcast(jnp.int32(0x5F3759DF) - (xi >> 1), jnp.float32)
    for _ in range(3):
        y = y * (1.5 - 0.5 * x * y * y)
    return y


def _sc_body(pop, acc, loc1, loc2, cf, mu, mv, spread, mutch, mask01, surv,
             out, acc_v, l1v, l2v, cfv, muv, i1v, i2v, p1v, p2v, mvv, outv,
             spv, mcv, mkv, survv, elitev, sem):
    wid = lax.axis_index("s") * 2 + lax.axis_index("c")
    base = wid * RPW
    pltpu.sync_copy(acc, acc_v)
    pltpu.sync_copy(spread, spv)
    pltpu.sync_copy(mutch, mcv)

    for k in range(NCH):
        rbase = base + k * CH
        pltpu.sync_copy(loc1.at[pl.ds(rbase, CH)], l1v)
        pltpu.sync_copy(loc2.at[pl.ds(rbase, CH)], l2v)
        pltpu.sync_copy(cf.at[pl.ds(rbase, CH)], cfv)
        pltpu.sync_copy(mu.at[pl.ds(rbase, CH)], muv)
        pltpu.sync_copy(mv.at[pl.ds(rbase, CH)], mvv)
        for g in range(CH // 16):
            sl = pl.ds(g * 16, 16)
            i1v[sl] = _count_below(acc_v, l1v[sl])
            i2v[sl] = _count_below(acc_v, l2v[sl])
        pltpu.async_copy(pop.at[i1v], p1v, sem).wait()
        pltpu.async_copy(pop.at[i2v], p2v, sem).wait()

        spl = spv[:]
        mcl = mcv[:]

        def row_fn(r, _):
            rv = jnp.full((16,), r, jnp.int32)
            kf = (plsc.load_gather(cfv, [rv]) - 0.5) * spl
            mur = plsc.load_gather(muv, [rv])
            ssum = jnp.zeros((16,), jnp.float32)
            for c in range(G // 16):
                mvc = mvv[r, pl.ds(c * 16, 16)]
                ssum = ssum + mvc * mvc
            tot = jnp.sum(ssum)
            tv = jnp.full((16,), tot)
            mscale = jnp.where(mur <= mcl, _rsqrt16(tv) * spl, 0.0)
            for c in range(G // 16):
                cs = pl.ds(c * 16, 16)
                p1c = p1v[r, cs]
                p2c = p2v[r, cs]
                outv[r, cs] = p1c + (p2c - p1c) * kf + mvv[r, cs] * mscale
            return 0

        lax.fori_loop(0, CH, row_fn, 0)

        if k == 0:
            @pl.when(wid == 0)
            def _elite():
                pltpu.sync_copy(surv, survv)
                pltpu.sync_copy(mask01, mkv)
                pltpu.async_copy(pop.at[survv], elitev, sem).wait()

                def blend_fn(r, _):
                    rv = jnp.full((16,), r, jnp.int32)
                    m = plsc.load_gather(mkv, [rv])
                    for c in range(G // 16):
                        cs = pl.ds(c * 16, 16)
                        outv[r, cs] = elitev[r, cs] * m + outv[r, cs] * (1.0 - m)
                    return 0

                lax.fori_loop(0, NSURV, blend_fn, 0)

        pltpu.sync_copy(outv, out.at[pl.ds(rbase, CH)])


@functools.partial(
    pl.kernel,
    mesh=plsc.VectorSubcoreMesh(core_axis_name="c", subcore_axis_name="s"),
    out_type=jax.ShapeDtypeStruct((POP, G), jnp.float32),
    compiler_params=pltpu.CompilerParams(needs_layout_passes=False),
    scratch_types=[
        pltpu.VMEM((POP,), jnp.float32),
        pltpu.VMEM((CH,), jnp.float32),
        pltpu.VMEM((CH,), jnp.float32),
        pltpu.VMEM((CH,), jnp.float32),
        pltpu.VMEM((CH,), jnp.float32),
        pltpu.VMEM((CH,), jnp.int32),
        pltpu.VMEM((CH,), jnp.int32),
        pltpu.VMEM((CH, G), jnp.float32),
        pltpu.VMEM((CH, G), jnp.float32),
        pltpu.VMEM((CH, G), jnp.float32),
        pltpu.VMEM((CH, G), jnp.float32),
        pltpu.VMEM((16,), jnp.float32),
        pltpu.VMEM((16,), jnp.float32),
        pltpu.VMEM((NSURV,), jnp.float32),
        pltpu.VMEM((NSURV,), jnp.int32),
        pltpu.VMEM((NSURV, G), jnp.float32),
        pltpu.SemaphoreType.DMA,
    ],
)
def _sc_kernel(pop, acc, loc1, loc2, cf, mu, mv, spread, mutch, mask01, surv,
               out, *scratch):
    _sc_body(pop, acc, loc1, loc2, cf, mu, mv, spread, mutch, mask01, surv,
             out, *scratch)


def kernel(population, scores, loc1, loc2, cf, mut_u, mut_vec, generation,
           n_survivors):
    scores_n = scores / jnp.sum(scores)
    acc = jnp.cumsum(scores_n)
    acc = acc.at[-1].set(1.0)
    gen = jnp.asarray(generation, jnp.float32)
    spread = jnp.exp(-gen / 10.0)
    mut_ch = 0.1 * jnp.exp(-gen / 10.0)
    spread_v = jnp.full((16,), spread, jnp.float32)
    mutch_v = jnp.full((16,), mut_ch, jnp.float32)
    mask01 = (jnp.arange(NSURV) < n_survivors).astype(jnp.float32)
    surv = _survivor_indices(scores_n)
    return _sc_kernel(
        population, acc,
        loc1.reshape(POP), loc2.reshape(POP),
        cf.reshape(POP), mut_u.reshape(POP), mut_vec,
        spread_v, mutch_v, mask01, surv,
    )


# speedup vs baseline: 2.8689x; 2.5025x over previous
import functools

import jax
import jax.numpy as jnp
from jax import lax
from jax.experimental import pallas as pl
from jax.experimental.pallas import tpu as pltpu
from jax.experimental.pallas import tpu_sc as plsc

POP = 8192
G = 256
NSURV = 64
NW = 32
RPW = POP // NW
CH = 64
NCH = RPW // CH


_BLK = 512
_NCAND = (POP // _BLK) * NSURV


def _rank_col(vrow, vcol, irow, icol):
    gt = vrow > vcol
    eqlt = (vrow == vcol) & (irow < icol)
    return jnp.sum(jnp.where(gt | eqlt, 1.0, 0.0), axis=1, keepdims=True)


def _onehot_compact_cols(rank_col, vals_col, nsel):
    n = rank_col.shape[0]
    rvec = lax.broadcasted_iota(jnp.int32, (n, nsel), 1).astype(jnp.float32)
    hit = jnp.where(rank_col == rvec, 1.0, 0.0)
    return jnp.sum(hit * vals_col, axis=0)


def _stage1_body(snr_ref, snc_ref, cval_ref, cidx_ref):
    b = pl.program_id(0)
    vrow = snr_ref[...]
    vcol = snc_ref[...]
    irow = lax.broadcasted_iota(jnp.int32, (1, _BLK), 1)
    icol = lax.broadcasted_iota(jnp.int32, (_BLK, 1), 0)
    lrank = _rank_col(vrow, vcol, irow, icol)
    ivals = (b * _BLK + icol).astype(jnp.float32)
    cval_ref[...] = _onehot_compact_cols(lrank, vcol, NSURV).reshape(
        1, 1, NSURV)
    cidx_ref[...] = _onehot_compact_cols(lrank, ivals, NSURV).reshape(
        1, 1, NSURV)


def _stage2_body(vr_ref, vc_ref, ir_ref, ic_ref, surv_ref):
    grank = _rank_col(vr_ref[...], vc_ref[...], ir_ref[...], ic_ref[...])
    surv_ref[...] = _onehot_compact_cols(grank, ic_ref[...], 128).reshape(
        1, 128)


def _survivor_indices(sn):
    nblk = POP // _BLK
    cval, cidx = pl.pallas_call(
        _stage1_body,
        grid=(nblk,),
        in_specs=[
            pl.BlockSpec((1, _BLK), lambda b: (0, b)),
            pl.BlockSpec((_BLK, 1), lambda b: (b, 0)),
        ],
        out_specs=[
            pl.BlockSpec((1, 1, NSURV), lambda b: (b, 0, 0)),
            pl.BlockSpec((1, 1, NSURV), lambda b: (b, 0, 0)),
        ],
        out_shape=[
            jax.ShapeDtypeStruct((nblk, 1, NSURV), jnp.float32),
            jax.ShapeDtypeStruct((nblk, 1, NSURV), jnp.float32),
        ],
    )(sn.reshape(1, POP), sn.reshape(POP, 1))
    surv = pl.pallas_call(
        _stage2_body,
        out_shape=jax.ShapeDtypeStruct((1, 128), jnp.float32),
    )(cval.reshape(1, _NCAND), cval.reshape(_NCAND, 1),
      cidx.reshape(1, _NCAND), cidx.reshape(_NCAND, 1))
    return surv[0, :NSURV].astype(jnp.int32)


_STEPS = (4096, 2048, 1024, 512, 256, 128, 64, 32, 16, 8, 4, 2, 1)


def _count_below(acc_v, q):
    pos = jnp.zeros((16,), jnp.int32)
    for s in _STEPS:
        cand = pos + s
        val = plsc.load_gather(acc_v, [cand - 1])
        pos = jnp.where(val < q, cand, pos)
    cnt = jnp.maximum(pos - 8, 0)
    for o in range(16):
        j = pos + (o - 8)
        jc = jnp.clip(j, 0, POP - 1)
        av = plsc.load_gather(acc_v, [jc])
        cnt = cnt + jnp.where((j >= 0) & (av < q), 1, 0)
    return cnt


def _rsqrt16(x):
    xi = plsc.bitcast(x, jnp.int32)
    y = plsc.bitcast(jnp.int32(0x5F3759DF) - (xi >> 1), jnp.float32)
    for _ in range(3):
        y = y * (1.5 - 0.5 * x * y * y)
    return y


def _sc_body(pop, acc, loc1, loc2, cf, mu, mv, spread, mutch, mask01, surv,
             out, acc_v, l1v, l2v, cfv, muv, i1v, i2v, p1v, p2v, mvv, outv,
             spv, mcv, mkv, survv, elitev, sem):
    wid = lax.axis_index("s") * 2 + lax.axis_index("c")
    base = wid * RPW
    pltpu.sync_copy(acc, acc_v)
    pltpu.sync_copy(spread, spv)
    pltpu.sync_copy(mutch, mcv)

    for k in range(NCH):
        rbase = base + k * CH
        pltpu.sync_copy(loc1.at[pl.ds(rbase, CH)], l1v)
        pltpu.sync_copy(loc2.at[pl.ds(rbase, CH)], l2v)
        pltpu.sync_copy(cf.at[pl.ds(rbase, CH)], cfv)
        pltpu.sync_copy(mu.at[pl.ds(rbase, CH)], muv)
        pltpu.sync_copy(mv.at[pl.ds(rbase, CH)], mvv)
        for g in range(CH // 16):
            sl = pl.ds(g * 16, 16)
            i1v[sl] = _count_below(acc_v, l1v[sl])
            i2v[sl] = _count_below(acc_v, l2v[sl])
        pltpu.async_copy(pop.at[i1v], p1v, sem).wait()
        pltpu.async_copy(pop.at[i2v], p2v, sem).wait()

        spl = spv[:]
        mcl = mcv[:]

        def row_fn(r, _):
            rv = jnp.full((16,), r, jnp.int32)
            kf = (plsc.load_gather(cfv, [rv]) - 0.5) * spl
            mur = plsc.load_gather(muv, [rv])
            ssum = jnp.zeros((16,), jnp.float32)
            for c in range(G // 16):
                mvc = mvv[r, pl.ds(c * 16, 16)]
                ssum = ssum + mvc * mvc
            tot = jnp.sum(ssum)
            tv = jnp.full((16,), tot)
            mscale = jnp.where(mur <= mcl, _rsqrt16(tv) * spl, 0.0)
            for c in range(G // 16):
                cs = pl.ds(c * 16, 16)
                p1c = p1v[r, cs]
                p2c = p2v[r, cs]
                outv[r, cs] = p1c + (p2c - p1c) * kf + mvv[r, cs] * mscale
            return 0

        lax.fori_loop(0, CH, row_fn, 0)

        if k == 0:
            @pl.when(wid == 0)
            def _elite():
                pltpu.sync_copy(surv, survv)
                pltpu.sync_copy(mask01, mkv)
                pltpu.async_copy(pop.at[survv], elitev, sem).wait()

                def blend_fn(r, _):
                    rv = jnp.full((16,), r, jnp.int32)
                    m = plsc.load_gather(mkv, [rv])
                    for c in range(G // 16):
                        cs = pl.ds(c * 16, 16)
                        outv[r, cs] = elitev[r, cs] * m + outv[r, cs] * (1.0 - m)
                    return 0

                lax.fori_loop(0, NSURV, blend_fn, 0)

        pltpu.sync_copy(outv, out.at[pl.ds(rbase, CH)])


@functools.partial(
    pl.kernel,
    mesh=plsc.VectorSubcoreMesh(core_axis_name="c", subcore_axis_name="s"),
    out_type=jax.ShapeDtypeStruct((POP, G), jnp.float32),
    compiler_params=pltpu.CompilerParams(needs_layout_passes=False),
    scratch_types=[
        pltpu.VMEM((POP,), jnp.float32),
        pltpu.VMEM((CH,), jnp.float32),
        pltpu.VMEM((CH,), jnp.float32),
        pltpu.VMEM((CH,), jnp.float32),
        pltpu.VMEM((CH,), jnp.float32),
        pltpu.VMEM((CH,), jnp.int32),
        pltpu.VMEM((CH,), jnp.int32),
        pltpu.VMEM((CH, G), jnp.float32),
        pltpu.VMEM((CH, G), jnp.float32),
        pltpu.VMEM((CH, G), jnp.float32),
        pltpu.VMEM((CH, G), jnp.float32),
        pltpu.VMEM((16,), jnp.float32),
        pltpu.VMEM((16,), jnp.float32),
        pltpu.VMEM((NSURV,), jnp.float32),
        pltpu.VMEM((NSURV,), jnp.int32),
        pltpu.VMEM((NSURV, G), jnp.float32),
        pltpu.SemaphoreType.DMA,
    ],
)
def _sc_kernel(pop, acc, loc1, loc2, cf, mu, mv, spread, mutch, mask01, surv,
               out, *scratch):
    _sc_body(pop, acc, loc1, loc2, cf, mu, mv, spread, mutch, mask01, surv,
             out, *scratch)


def kernel(population, scores, loc1, loc2, cf, mut_u, mut_vec, generation,
           n_survivors):
    scores_n = scores / jnp.sum(scores)
    acc = jnp.cumsum(scores_n)
    acc = acc.at[-1].set(1.0)
    gen = jnp.asarray(generation, jnp.float32)
    spread = jnp.exp(-gen / 10.0)
    mut_ch = 0.1 * jnp.exp(-gen / 10.0)
    spread_v = jnp.full((16,), spread, jnp.float32)
    mutch_v = jnp.full((16,), mut_ch, jnp.float32)
    mask01 = (jnp.arange(NSURV) < n_survivors).astype(jnp.float32)
    surv = _survivor_indices(scores_n)
    return _sc_kernel(
        population, acc,
        loc1.reshape(POP), loc2.reshape(POP),
        cf.reshape(POP), mut_u.reshape(POP), mut_vec,
        spread_v, mutch_v, mask01, surv,
    )
